# baseline (speedup 1.0000x reference)
import functools

import jax
import jax.numpy as jnp
from jax import lax
from jax.experimental import pallas as pl
from jax.experimental.pallas import tpu as pltpu
from jax.experimental.pallas import tpu_sc as plsc

TOKEN_DIM = 1024
HID = 32


def _sc_gather(table, idx):
    B = idx.shape[0]
    D = table.shape[1]
    W = D // 2
    L = 16
    info = plsc.get_sparse_core_info()
    nw = info.num_cores * info.num_subcores
    b_per_w = B // nw
    CH = 32
    n_ch = b_per_w // CH
    mesh = plsc.VectorSubcoreMesh(core_axis_name="c", subcore_axis_name="s")

    @functools.partial(
        pl.kernel,
        mesh=mesh,
        compiler_params=pltpu.CompilerParams(needs_layout_passes=False),
        out_type=jax.ShapeDtypeStruct((B, W), jnp.uint32),
        scratch_types=[
            pltpu.VMEM((b_per_w,), jnp.int32),
            pltpu.VMEM((CH, D), jnp.float32),
            pltpu.VMEM((CH, D), jnp.float32),
            pltpu.VMEM((CH, W), jnp.uint32),
            pltpu.VMEM((CH, W), jnp.uint32),
            pltpu.SemaphoreType.DMA,
            pltpu.SemaphoreType.DMA,
            pltpu.SemaphoreType.DMA,
            pltpu.SemaphoreType.DMA,
        ],
    )
    def k(table_hbm, idx_hbm, out_hbm, idx_v, rows0, rows1, pk0, pk1,
          sem0, sem1, wsem0, wsem1):
        wid = lax.axis_index("s") * info.num_cores + lax.axis_index("c")
        base = wid * b_per_w
        pltpu.sync_copy(idx_hbm.at[pl.ds(base, b_per_w)], idx_v)
        rbufs = (rows0, rows1)
        pbufs = (pk0, pk1)
        gsems = (sem0, sem1)
        wsems = (wsem0, wsem1)
        himask = jnp.full((L,), 0xFFFF0000, dtype=jnp.uint32)

        def gather(c):
            return pltpu.make_async_copy(
                table_hbm.at[idx_v.at[pl.ds(c * CH, CH)]],
                rbufs[c % 2], gsems[c % 2])

        def writeout(c):
            return pltpu.make_async_copy(
                pbufs[c % 2], out_hbm.at[pl.ds(base + c * CH, CH)],
                wsems[c % 2])

        def pack(c):
            rows_v = rbufs[c % 2]
            pk_v = pbufs[c % 2]

            def row_body(r, _):
                for kk in range(W // L):
                    a = plsc.bitcast(rows_v[r, pl.ds(kk * L, L)], jnp.uint32)
                    b = plsc.bitcast(rows_v[r, pl.ds(W + kk * L, L)],
                                     jnp.uint32)
                    pk_v[r, pl.ds(kk * L, L)] = (a >> 16) | (b & himask)
                return 0

            lax.fori_loop(0, CH, row_body, 0)

        gather(0).start()
        for c in range(n_ch):
            if c + 1 < n_ch:
                gather(c + 1).start()
            gather(c).wait()
            if c >= 2:
                writeout(c - 2).wait()
            pack(c)
            writeout(c).start()
        writeout(n_ch - 2).wait()
        writeout(n_ch - 1).wait()

    return k(table, idx)


def _tc_body(td_ref, nv_ref, sm_ref, vm_ref, g_ref,
             dW1, db1, dW2, db2, vW1, vb1, vW2, vb2,
             tst, cpf, vpf, out_ref):
    D = TOKEN_DIM
    blk = td_ref.shape[0]

    td = td_ref[...]
    h_t = jnp.tanh(td * dW1[...] + db1[...])
    emb_t = jnp.dot(h_t, dW2[...],
                    preferred_element_type=jnp.float32) + db2[...]

    nv = nv_ref[...]
    h_v = jnp.tanh(nv * vW1[...] + vb1[...])
    emb_v = jnp.dot(h_v, vW2[...],
                    preferred_element_type=jnp.float32) + vb2[...]

    tmask = (sm_ref[...] > 0.0) & (td != 0.0)
    vmask = vm_ref[...] > 0.0

    ts_row = jnp.broadcast_to(tst[...], (blk, D))
    vp_row = jnp.broadcast_to(vpf[...], (blk, D))

    W = D // 2
    w = g_ref[...]
    g_lo = lax.bitcast_convert_type(w << 16, jnp.float32)
    g_hi = lax.bitcast_convert_type(w & jnp.uint32(0xFFFF0000), jnp.float32)

    out_ref[:, 0 * D:1 * D] = jnp.where(tmask, emb_t, ts_row)
    out_ref[:, 1 * D:2 * D] = jnp.broadcast_to(cpf[...], (blk, D))
    out_ref[:, 2 * D:2 * D + W] = g_lo
    out_ref[:, 2 * D + W:3 * D] = g_hi
    out_ref[:, 3 * D:4 * D] = vp_row
    out_ref[:, 4 * D:5 * D] = jnp.where(vmask, emb_v, vp_row)


def _tc_assemble(td, nv, sm, vm, g,
                 dW1, db1, dW2, db2, vW1, vb1, vW2, vb2,
                 tst, cpf, vpf):
    N = td.shape[0]
    D = TOKEN_DIM
    BLK = 1024
    grid = (N // BLK,)

    col = lambda i: (i, 0)
    rep = lambda i: (0, 0)
    specs = [
        pl.BlockSpec((BLK, 1), col),
        pl.BlockSpec((BLK, 1), col),
        pl.BlockSpec((BLK, 1), col),
        pl.BlockSpec((BLK, 1), col),
        pl.BlockSpec((BLK, D // 2), col),
        pl.BlockSpec((1, HID), rep),
        pl.BlockSpec((1, HID), rep),
        pl.BlockSpec((HID, D), rep),
        pl.BlockSpec((1, D), rep),
        pl.BlockSpec((1, HID), rep),
        pl.BlockSpec((1, HID), rep),
        pl.BlockSpec((HID, D), rep),
        pl.BlockSpec((1, D), rep),
        pl.BlockSpec((1, D), rep),
        pl.BlockSpec((1, D), rep),
        pl.BlockSpec((1, D), rep),
    ]
    return pl.pallas_call(
        _tc_body,
        grid=grid,
        in_specs=specs,
        out_specs=pl.BlockSpec((BLK, 5 * D), col),
        out_shape=jax.ShapeDtypeStruct((N, 5 * D), jnp.float32),
    )(td, nv, sm, vm, g,
      dW1, db1, dW2, db2, vW1, vb1, vW2, vb2, tst, cpf, vpf)


def kernel(static_mask, code, numerical_value, time_delta_days,
           numerical_value_mask, mask, code_table,
           date_W1, date_b1, date_W2, date_b2,
           val_W1, val_b1, val_W2, val_b2,
           ts_token, code_prefix, val_prefix):
    N = code.shape[0]
    g = _sc_gather(code_table, code.astype(jnp.int32))

    col = lambda a: a.astype(jnp.float32).reshape(N, 1)
    row = lambda a: a.reshape(1, -1)
    return _tc_assemble(
        col(time_delta_days), col(numerical_value),
        col(static_mask), col(numerical_value_mask), g,
        date_W1, row(date_b1), date_W2, row(date_b2),
        val_W1, row(val_b1), val_W2, row(val_b2),
        row(ts_token), row(code_prefix), row(val_prefix))


# speedup vs baseline: 1.2033x; 1.2033x over previous
import functools

import jax
import jax.numpy as jnp
from jax import lax
from jax.experimental import pallas as pl
from jax.experimental.pallas import tpu as pltpu
from jax.experimental.pallas import tpu_sc as plsc

TOKEN_DIM = 1024
HID = 32


def _sc_gather(table, idx):
    B = idx.shape[0]
    D = table.shape[1]
    W = D // 2
    L = 16
    info = plsc.get_sparse_core_info()
    nw = info.num_cores * info.num_subcores
    b_per_w = B // nw
    CH = 32
    n_ch = b_per_w // CH
    mesh = plsc.VectorSubcoreMesh(core_axis_name="c", subcore_axis_name="s")

    @functools.partial(
        pl.kernel,
        mesh=mesh,
        compiler_params=pltpu.CompilerParams(needs_layout_passes=False),
        out_type=jax.ShapeDtypeStruct((B, W), jnp.uint32),
        scratch_types=[
            pltpu.VMEM((b_per_w,), jnp.int32),
            pltpu.VMEM((CH, D), jnp.float32),
            pltpu.VMEM((CH, D), jnp.float32),
            pltpu.VMEM((CH, W), jnp.uint32),
            pltpu.VMEM((CH, W), jnp.uint32),
            pltpu.SemaphoreType.DMA,
            pltpu.SemaphoreType.DMA,
            pltpu.SemaphoreType.DMA,
            pltpu.SemaphoreType.DMA,
        ],
    )
    def k(table_hbm, idx_hbm, out_hbm, idx_v, rows0, rows1, pk0, pk1,
          sem0, sem1, wsem0, wsem1):
        wid = lax.axis_index("s") * info.num_cores + lax.axis_index("c")
        base = wid * b_per_w
        pltpu.sync_copy(idx_hbm.at[pl.ds(base, b_per_w)], idx_v)
        rbufs = (rows0, rows1)
        pbufs = (pk0, pk1)
        gsems = (sem0, sem1)
        wsems = (wsem0, wsem1)
        himask = jnp.full((L,), 0xFFFF0000, dtype=jnp.uint32)

        def gather(c):
            return pltpu.make_async_copy(
                table_hbm.at[idx_v.at[pl.ds(c * CH, CH)]],
                rbufs[c % 2], gsems[c % 2])

        def writeout(c):
            return pltpu.make_async_copy(
                pbufs[c % 2], out_hbm.at[pl.ds(base + c * CH, CH)],
                wsems[c % 2])

        def pack(c):
            rows_v = rbufs[c % 2]
            pk_v = pbufs[c % 2]

            @plsc.parallel_loop(0, CH, unroll=2)
            def row_body(r):
                for kk in range(W // L):
                    a = plsc.bitcast(rows_v[r, pl.ds(kk * L, L)], jnp.uint32)
                    b = plsc.bitcast(rows_v[r, pl.ds(W + kk * L, L)],
                                     jnp.uint32)
                    pk_v[r, pl.ds(kk * L, L)] = (a >> 16) | (b & himask)

        gather(0).start()
        for c in range(n_ch):
            if c + 1 < n_ch:
                gather(c + 1).start()
            gather(c).wait()
            if c >= 2:
                writeout(c - 2).wait()
            pack(c)
            writeout(c).start()
        writeout(n_ch - 2).wait()
        writeout(n_ch - 1).wait()

    return k(table, idx)


def _tc_body(td_ref, nv_ref, sm_ref, vm_ref, g_ref,
             dW1, db1, dW2, db2, vW1, vb1, vW2, vb2,
             tst, cpf, vpf, out_ref):
    D = TOKEN_DIM
    blk = td_ref.shape[0]

    td = td_ref[...]
    h_t = jnp.tanh(td * dW1[...] + db1[...])
    emb_t = jnp.dot(h_t, dW2[...],
                    preferred_element_type=jnp.float32) + db2[...]

    nv = nv_ref[...]
    h_v = jnp.tanh(nv * vW1[...] + vb1[...])
    emb_v = jnp.dot(h_v, vW2[...],
                    preferred_element_type=jnp.float32) + vb2[...]

    tmask = (sm_ref[...] > 0.0) & (td != 0.0)
    vmask = vm_ref[...] > 0.0

    ts_row = jnp.broadcast_to(tst[...], (blk, D))
    vp_row = jnp.broadcast_to(vpf[...], (blk, D))

    W = D // 2
    w = g_ref[...]
    g_lo = lax.bitcast_convert_type(w << 16, jnp.float32)
    g_hi = lax.bitcast_convert_type(w & jnp.uint32(0xFFFF0000), jnp.float32)

    out_ref[:, 0 * D:1 * D] = jnp.where(tmask, emb_t, ts_row)
    out_ref[:, 1 * D:2 * D] = jnp.broadcast_to(cpf[...], (blk, D))
    out_ref[:, 2 * D:2 * D + W] = g_lo
    out_ref[:, 2 * D + W:3 * D] = g_hi
    out_ref[:, 3 * D:4 * D] = vp_row
    out_ref[:, 4 * D:5 * D] = jnp.where(vmask, emb_v, vp_row)


def _tc_assemble(td, nv, sm, vm, g,
                 dW1, db1, dW2, db2, vW1, vb1, vW2, vb2,
                 tst, cpf, vpf):
    N = td.shape[0]
    D = TOKEN_DIM
    BLK = 1024
    grid = (N // BLK,)

    col = lambda i: (i, 0)
    rep = lambda i: (0, 0)
    specs = [
        pl.BlockSpec((BLK, 1), col),
        pl.BlockSpec((BLK, 1), col),
        pl.BlockSpec((BLK, 1), col),
        pl.BlockSpec((BLK, 1), col),
        pl.BlockSpec((BLK, D // 2), col),
        pl.BlockSpec((1, HID), rep),
        pl.BlockSpec((1, HID), rep),
        pl.BlockSpec((HID, D), rep),
        pl.BlockSpec((1, D), rep),
        pl.BlockSpec((1, HID), rep),
        pl.BlockSpec((1, HID), rep),
        pl.BlockSpec((HID, D), rep),
        pl.BlockSpec((1, D), rep),
        pl.BlockSpec((1, D), rep),
        pl.BlockSpec((1, D), rep),
        pl.BlockSpec((1, D), rep),
    ]
    return pl.pallas_call(
        _tc_body,
        grid=grid,
        in_specs=specs,
        out_specs=pl.BlockSpec((BLK, 5 * D), col),
        out_shape=jax.ShapeDtypeStruct((N, 5 * D), jnp.float32),
    )(td, nv, sm, vm, g,
      dW1, db1, dW2, db2, vW1, vb1, vW2, vb2, tst, cpf, vpf)


def kernel(static_mask, code, numerical_value, time_delta_days,
           numerical_value_mask, mask, code_table,
           date_W1, date_b1, date_W2, date_b2,
           val_W1, val_b1, val_W2, val_b2,
           ts_token, code_prefix, val_prefix):
    N = code.shape[0]
    g = _sc_gather(code_table, code.astype(jnp.int32))

    col = lambda a: a.astype(jnp.float32).reshape(N, 1)
    row = lambda a: a.reshape(1, -1)
    return _tc_assemble(
        col(time_delta_days), col(numerical_value),
        col(static_mask), col(numerical_value_mask), g,
        date_W1, row(date_b1), date_W2, row(date_b2),
        val_W1, row(val_b1), val_W2, row(val_b2),
        row(ts_token), row(code_prefix), row(val_prefix))
